# baseline (device time: 90035 ns/iter reference)
import jax
import jax.numpy as jnp
from jax import lax
from jax.experimental import pallas as pl
from jax.experimental.pallas import tpu as pltpu

N_DEV = 8
B = 16
NB = 128
BS = 16
H = 16
D = 64
PAGES = 128
TOK = PAGES * BS
NEG = -1e30


def kernel(Q, K, V, bt, lens):
    lens2 = lens.reshape(B, 1)

    def body(q_ref, k_ref, v_ref, bt_ref, lens_ref, out_ref,
             gbuf, send_sems, recv_sems):
        my_pos = lax.axis_index("i")
        my_lo = my_pos * PAGES

        btv = bt_ref[...]
        ln = lens_ref[...]
        slot = lax.broadcasted_iota(jnp.int32, (B, NB), 1)
        btv = jnp.where(slot < ln, btv, -1)
        page_ids = my_lo + lax.broadcasted_iota(jnp.int32, (B, NB, PAGES), 2)
        w = jnp.sum((btv[:, :, None] == page_ids).astype(jnp.float32), axis=1)
        w_tok = jnp.broadcast_to(w[:, :, None], (B, PAGES, BS)).reshape(B, TOK)
        has_key = w_tok > 0.0

        kv = k_ref[...].reshape(TOK, H, D)
        vv = v_ref[...].reshape(TOK, H, D)
        scale = D ** -0.5

        for h in range(H):
            q_h = q_ref[:, 0, h, :]
            k_h = kv[:, h, :]
            v_h = vv[:, h, :]
            s = lax.dot_general(
                q_h, k_h, (((1,), (1,)), ((), ())),
                preferred_element_type=jnp.float32,
            ) * scale
            s = jnp.where(has_key, s, NEG)
            m_h = jnp.max(s, axis=1, keepdims=True)
            p_h = jnp.exp(s - m_h) * w_tok
            l_h = jnp.sum(p_h, axis=1, keepdims=True)
            acc_h = lax.dot_general(
                p_h, v_h, (((1,), (0,)), ((), ())),
                preferred_element_type=jnp.float32,
            )
            gbuf[my_pos, 0, :, h, :] = acc_h
            gbuf[my_pos, 1, :, h, :] = jnp.broadcast_to(m_h, (B, D))
            gbuf[my_pos, 2, :, h, :] = jnp.broadcast_to(l_h, (B, D))

        barrier_sem = pltpu.get_barrier_semaphore()
        for k in range(1, N_DEV):
            pl.semaphore_signal(
                barrier_sem, inc=1,
                device_id=(lax.rem(my_pos + k, N_DEV),),
                device_id_type=pl.DeviceIdType.MESH,
            )
        pl.semaphore_wait(barrier_sem, N_DEV - 1)

        rdmas = []
        for k in range(1, N_DEV):
            rdma = pltpu.make_async_remote_copy(
                src_ref=gbuf.at[my_pos],
                dst_ref=gbuf.at[my_pos],
                send_sem=send_sems.at[k - 1],
                recv_sem=recv_sems.at[k - 1],
                device_id=(lax.rem(my_pos + k, N_DEV),),
                device_id_type=pl.DeviceIdType.MESH,
            )
            rdma.start()
            rdmas.append(rdma)
        for rdma in rdmas:
            rdma.wait()

        g = gbuf[...]
        acc = g[:, 0]
        m = g[:, 1]
        l = g[:, 2]
        mx = jnp.max(m, axis=0)
        sc = jnp.exp(m - mx[None])
        num = jnp.sum(sc * acc, axis=0)
        den = jnp.sum(sc * l, axis=0)
        out_ref[:, 0, :, :] = num / den

    return pl.pallas_call(
        body,
        out_shape=jax.ShapeDtypeStruct((B, 1, H, D), jnp.float32),
        in_specs=[
            pl.BlockSpec(memory_space=pltpu.VMEM),
            pl.BlockSpec(memory_space=pltpu.VMEM),
            pl.BlockSpec(memory_space=pltpu.VMEM),
            pl.BlockSpec(memory_space=pltpu.VMEM),
            pl.BlockSpec(memory_space=pltpu.VMEM),
        ],
        out_specs=pl.BlockSpec(memory_space=pltpu.VMEM),
        scratch_shapes=[
            pltpu.VMEM((N_DEV, 3, B, H, D), jnp.float32),
            pltpu.SemaphoreType.DMA((N_DEV - 1,)),
            pltpu.SemaphoreType.DMA((N_DEV - 1,)),
        ],
        compiler_params=pltpu.CompilerParams(collective_id=0),
    )(Q, K, V, bt, lens2)


# device time: 50250 ns/iter; 1.7917x vs baseline; 1.7917x over previous
import jax
import jax.numpy as jnp
from jax import lax
from jax.experimental import pallas as pl
from jax.experimental.pallas import tpu as pltpu

N_DEV = 8
B = 16
NB = 128
BS = 16
H = 16
D = 64
PAGES = 128
TOK = PAGES * BS
HD = H * D
NEG = -1e30


def kernel(Q, K, V, bt, lens):
    q2 = Q.reshape(B, HD)
    k2 = K.reshape(TOK, HD)
    v2 = V.reshape(TOK, HD)
    lens2 = lens.reshape(B, 1)

    def body(q_ref, k_ref, v_ref, bt_ref, lens_ref, out_ref,
             gbuf, send_sems, recv_sems):
        my_pos = lax.axis_index("i")
        my_lo = my_pos * PAGES

        btv = bt_ref[...]
        ln = lens_ref[...]
        slot = lax.broadcasted_iota(jnp.int32, (B, NB), 1)
        btv = jnp.where(slot < ln, btv, -1)
        page_ids = my_lo + lax.broadcasted_iota(jnp.int32, (B, NB, PAGES), 2)
        w = jnp.sum((btv[:, :, None] == page_ids).astype(jnp.float32), axis=1)
        w_tok = jnp.broadcast_to(w[:, :, None], (B, PAGES, BS)).reshape(B, TOK)
        has_key = w_tok > 0.0

        scale = D ** -0.5

        accs, ms, ls = [], [], []
        for h in range(H):
            sl = pl.ds(h * D, D)
            q_h = q_ref[:, sl]
            k_h = k_ref[:, sl]
            v_h = v_ref[:, sl]
            s = lax.dot_general(
                q_h, k_h, (((1,), (1,)), ((), ())),
                preferred_element_type=jnp.float32,
            ) * scale
            s = jnp.where(has_key, s, NEG)
            m_h = jnp.max(s, axis=1, keepdims=True)
            p_h = jnp.exp(s - m_h) * w_tok
            l_h = jnp.sum(p_h, axis=1, keepdims=True)
            acc_h = lax.dot_general(
                p_h, v_h, (((1,), (0,)), ((), ())),
                preferred_element_type=jnp.float32,
            )
            accs.append(acc_h)
            ms.append(jnp.broadcast_to(m_h, (B, D)))
            ls.append(jnp.broadcast_to(l_h, (B, D)))
        gbuf[my_pos, 0] = jnp.concatenate(accs, axis=1)
        gbuf[my_pos, 1] = jnp.concatenate(ms, axis=1)
        gbuf[my_pos, 2] = jnp.concatenate(ls, axis=1)

        barrier_sem = pltpu.get_barrier_semaphore()
        for k in range(1, N_DEV):
            pl.semaphore_signal(
                barrier_sem, inc=1,
                device_id=(lax.rem(my_pos + k, N_DEV),),
                device_id_type=pl.DeviceIdType.MESH,
            )
        pl.semaphore_wait(barrier_sem, N_DEV - 1)

        rdmas = []
        for k in range(1, N_DEV):
            rdma = pltpu.make_async_remote_copy(
                src_ref=gbuf.at[my_pos],
                dst_ref=gbuf.at[my_pos],
                send_sem=send_sems.at[k - 1],
                recv_sem=recv_sems.at[k - 1],
                device_id=(lax.rem(my_pos + k, N_DEV),),
                device_id_type=pl.DeviceIdType.MESH,
            )
            rdma.start()
            rdmas.append(rdma)
        for rdma in rdmas:
            rdma.wait()

        g = gbuf[...]
        acc = g[:, 0]
        m = g[:, 1]
        l = g[:, 2]
        mx = jnp.max(m, axis=0)
        sc = jnp.exp(m - mx[None])
        num = jnp.sum(sc * acc, axis=0)
        den = jnp.sum(sc * l, axis=0)
        out_ref[:, 0, :, :] = (num / den).reshape(B, H, D)

    return pl.pallas_call(
        body,
        out_shape=jax.ShapeDtypeStruct((B, 1, H, D), jnp.float32),
        in_specs=[
            pl.BlockSpec(memory_space=pltpu.VMEM),
            pl.BlockSpec(memory_space=pltpu.VMEM),
            pl.BlockSpec(memory_space=pltpu.VMEM),
            pl.BlockSpec(memory_space=pltpu.VMEM),
            pl.BlockSpec(memory_space=pltpu.VMEM),
        ],
        out_specs=pl.BlockSpec(memory_space=pltpu.VMEM),
        scratch_shapes=[
            pltpu.VMEM((N_DEV, 3, B, HD), jnp.float32),
            pltpu.SemaphoreType.DMA((N_DEV - 1,)),
            pltpu.SemaphoreType.DMA((N_DEV - 1,)),
        ],
        compiler_params=pltpu.CompilerParams(collective_id=0),
    )(q2, k2, v2, bt, lens2)


# device time: 32655 ns/iter; 2.7572x vs baseline; 1.5388x over previous
import jax
import jax.numpy as jnp
from jax import lax
from jax.experimental import pallas as pl
from jax.experimental.pallas import tpu as pltpu

N_DEV = 8
B = 16
NB = 128
BS = 16
H = 16
D = 64
PAGES = 128
TOK = PAGES * BS
NEG = -1e30


def kernel(Q, K, V, bt, lens):
    lens2 = lens.reshape(B, 1)
    kt = K.transpose(1, 2, 3, 0)
    vt = V.transpose(1, 2, 3, 0)

    def body(q_ref, k_ref, v_ref, bt_ref, lens_ref, out_ref,
             kbuf, vbuf, gbuf, kv_sems, send_sems, recv_sems):
        my_pos = lax.axis_index("i")
        my_lo = my_pos * PAGES

        kdmas = [[None] * BS for _ in range(H)]
        vdmas = [[None] * BS for _ in range(H)]
        for h in range(H):
            for t in range(BS):
                lanes = pl.ds(t * PAGES, PAGES)
                kd = pltpu.make_async_copy(
                    k_ref.at[t, h], kbuf.at[h, :, lanes], kv_sems.at[0, h])
                vd = pltpu.make_async_copy(
                    v_ref.at[t, h], vbuf.at[h, :, lanes], kv_sems.at[1, h])
                kd.start()
                vd.start()
                kdmas[h][t] = kd
                vdmas[h][t] = vd

        btv = bt_ref[...]
        ln = lens_ref[...]
        slot = lax.broadcasted_iota(jnp.int32, (B, NB), 1)
        btv = jnp.where(slot < ln, btv, -1)
        page_ids = my_lo + lax.broadcasted_iota(jnp.int32, (B, NB, PAGES), 2)
        w = jnp.sum((btv[:, :, None] == page_ids).astype(jnp.float32), axis=1)
        w_tok = jnp.broadcast_to(w[:, None, :], (B, BS, PAGES)).reshape(B, TOK)
        has_key = w_tok > 0.0

        scale = D ** -0.5

        ms, ls = [], []
        for h in range(H):
            q_h = q_ref[:, 0, h, :]
            for t in range(BS):
                kdmas[h][t].wait()
            k_h = kbuf[h]
            s = lax.dot_general(
                q_h, k_h, (((1,), (0,)), ((), ())),
                preferred_element_type=jnp.float32,
            ) * scale
            s = jnp.where(has_key, s, NEG)
            m_h = jnp.max(s, axis=1, keepdims=True)
            p_h = jnp.exp(s - m_h) * w_tok
            l_h = jnp.sum(p_h, axis=1, keepdims=True)
            for t in range(BS):
                vdmas[h][t].wait()
            v_h = vbuf[h]
            acc_h = lax.dot_general(
                p_h, v_h, (((1,), (1,)), ((), ())),
                preferred_element_type=jnp.float32,
            )
            gbuf[my_pos, h] = acc_h
            ms.append(m_h)
            ls.append(l_h)
        stats = jnp.concatenate(
            ms + ls + [jnp.zeros((B, D - 2 * H), jnp.float32)], axis=1)
        gbuf[my_pos, H] = stats

        barrier_sem = pltpu.get_barrier_semaphore()
        for k in range(1, N_DEV):
            pl.semaphore_signal(
                barrier_sem, inc=1,
                device_id=(lax.rem(my_pos + k, N_DEV),),
                device_id_type=pl.DeviceIdType.MESH,
            )
        pl.semaphore_wait(barrier_sem, N_DEV - 1)

        rdmas = []
        for k in range(1, N_DEV):
            rdma = pltpu.make_async_remote_copy(
                src_ref=gbuf.at[my_pos],
                dst_ref=gbuf.at[my_pos],
                send_sem=send_sems.at[k - 1],
                recv_sem=recv_sems.at[k - 1],
                device_id=(lax.rem(my_pos + k, N_DEV),),
                device_id_type=pl.DeviceIdType.MESH,
            )
            rdma.start()
            rdmas.append(rdma)
        for rdma in rdmas:
            rdma.wait()

        g = gbuf[...]
        mT = g[:, H, :, 0:H]
        lT = g[:, H, :, H:2 * H]
        mx = jnp.max(mT, axis=0)
        sc = jnp.exp(mT - mx[None])
        den = jnp.sum(sc * lT, axis=0)
        scT = jnp.transpose(sc, (0, 2, 1))
        num = jnp.sum(scT[..., None] * g[:, :H], axis=0)
        res = num / jnp.transpose(den, (1, 0))[..., None]
        out_ref[:, 0, :, :] = jnp.transpose(res, (1, 0, 2))

    return pl.pallas_call(
        body,
        out_shape=jax.ShapeDtypeStruct((B, 1, H, D), jnp.float32),
        in_specs=[
            pl.BlockSpec(memory_space=pltpu.VMEM),
            pl.BlockSpec(memory_space=pltpu.HBM),
            pl.BlockSpec(memory_space=pltpu.HBM),
            pl.BlockSpec(memory_space=pltpu.VMEM),
            pl.BlockSpec(memory_space=pltpu.VMEM),
        ],
        out_specs=pl.BlockSpec(memory_space=pltpu.VMEM),
        scratch_shapes=[
            pltpu.VMEM((H, D, TOK), jnp.float32),
            pltpu.VMEM((H, D, TOK), jnp.float32),
            pltpu.VMEM((N_DEV, H + 1, B, D), jnp.float32),
            pltpu.SemaphoreType.DMA((2, H)),
            pltpu.SemaphoreType.DMA((N_DEV - 1,)),
            pltpu.SemaphoreType.DMA((N_DEV - 1,)),
        ],
        compiler_params=pltpu.CompilerParams(
            collective_id=0, vmem_limit_bytes=64 * 1024 * 1024),
    )(Q, kt, vt, bt, lens2)


# device time: 30437 ns/iter; 2.9581x vs baseline; 1.0729x over previous
import jax
import jax.numpy as jnp
from jax import lax
from jax.experimental import pallas as pl
from jax.experimental.pallas import tpu as pltpu

N_DEV = 8
B = 16
NB = 128
BS = 16
H = 16
D = 64
PAGES = 128
TOK = PAGES * BS
NEG = -1e30


def kernel(Q, K, V, bt, lens):
    lens2 = lens.reshape(1, B)
    kt = K.transpose(1, 2, 3, 0)
    vt = V.transpose(1, 2, 3, 0)

    def body(q_ref, k_ref, v_ref, bt_ref, lens_ref, out_ref,
             kbuf, vbuf, gbuf, kv_sems, send_sems, recv_sems):
        my_pos = lax.axis_index("i")
        my_lo = my_pos * PAGES

        kdmas = [[None] * BS for _ in range(H)]
        vdmas = [[None] * BS for _ in range(H)]
        for h in range(H):
            for t in range(BS):
                lanes = pl.ds(t * PAGES, PAGES)
                kd = pltpu.make_async_copy(
                    k_ref.at[t, h], kbuf.at[h, :, lanes], kv_sems.at[0, h])
                vd = pltpu.make_async_copy(
                    v_ref.at[t, h], vbuf.at[h, :, lanes], kv_sems.at[1, h])
                kd.start()
                vd.start()
                kdmas[h][t] = kd
                vdmas[h][t] = vd

        barrier_sem = pltpu.get_barrier_semaphore()
        for k in range(1, N_DEV):
            pl.semaphore_signal(
                barrier_sem, inc=1,
                device_id=(lax.rem(my_pos + k, N_DEV),),
                device_id_type=pl.DeviceIdType.MESH,
            )

        btv = bt_ref[...]
        ln = jnp.transpose(lens_ref[...], (1, 0))
        slot = lax.broadcasted_iota(jnp.int32, (B, NB), 1)
        btv = jnp.where(slot < ln, btv, -1)
        page_ids = my_lo + lax.broadcasted_iota(jnp.int32, (B, NB, PAGES), 2)
        w = jnp.sum((btv[:, :, None] == page_ids).astype(jnp.float32), axis=1)
        w_tok = jnp.broadcast_to(w[:, None, :], (B, BS, PAGES)).reshape(B, TOK)
        has_key = w_tok > 0.0

        scale = D ** -0.5

        ms, ls = [], []
        for h in range(H):
            q_h = q_ref[:, 0, h, :]
            for t in range(BS):
                kdmas[h][t].wait()
            k_h = kbuf[h]
            s = lax.dot_general(
                q_h, k_h, (((1,), (0,)), ((), ())),
                preferred_element_type=jnp.float32,
            ) * scale
            s = jnp.where(has_key, s, NEG)
            m_h = jnp.max(s, axis=1, keepdims=True)
            p_h = jnp.exp(s - m_h) * w_tok
            l_h = jnp.sum(p_h, axis=1, keepdims=True)
            for t in range(BS):
                vdmas[h][t].wait()
            v_h = vbuf[h]
            acc_h = lax.dot_general(
                p_h, v_h, (((1,), (1,)), ((), ())),
                preferred_element_type=jnp.float32,
            )
            gbuf[my_pos, h] = acc_h
            ms.append(m_h)
            ls.append(l_h)
        stats = jnp.concatenate(
            ms + ls + [jnp.zeros((B, D - 2 * H), jnp.float32)], axis=1)
        gbuf[my_pos, H] = stats

        pl.semaphore_wait(barrier_sem, N_DEV - 1)

        rdmas = []
        for k in range(1, N_DEV):
            rdma = pltpu.make_async_remote_copy(
                src_ref=gbuf.at[my_pos],
                dst_ref=gbuf.at[my_pos],
                send_sem=send_sems.at[k - 1],
                recv_sem=recv_sems.at[k - 1],
                device_id=(lax.rem(my_pos + k, N_DEV),),
                device_id_type=pl.DeviceIdType.MESH,
            )
            rdma.start()
            rdmas.append(rdma)
        for rdma in rdmas:
            rdma.wait()

        g = gbuf[...]
        mT = g[:, H, :, 0:H]
        lT = g[:, H, :, H:2 * H]
        mx = jnp.max(mT, axis=0)
        sc = jnp.exp(mT - mx[None])
        den = jnp.sum(sc * lT, axis=0)
        scT = jnp.transpose(sc, (0, 2, 1))
        num = jnp.sum(scT[..., None] * g[:, :H], axis=0)
        res = num / jnp.transpose(den, (1, 0))[..., None]
        out_ref[:, 0, :, :] = jnp.transpose(res, (1, 0, 2))

    return pl.pallas_call(
        body,
        out_shape=jax.ShapeDtypeStruct((B, 1, H, D), jnp.float32),
        in_specs=[
            pl.BlockSpec(memory_space=pltpu.VMEM),
            pl.BlockSpec(memory_space=pltpu.HBM),
            pl.BlockSpec(memory_space=pltpu.HBM),
            pl.BlockSpec(memory_space=pltpu.VMEM),
            pl.BlockSpec(memory_space=pltpu.VMEM),
        ],
        out_specs=pl.BlockSpec(memory_space=pltpu.VMEM),
        scratch_shapes=[
            pltpu.VMEM((H, D, TOK), jnp.float32),
            pltpu.VMEM((H, D, TOK), jnp.float32),
            pltpu.VMEM((N_DEV, H + 1, B, D), jnp.float32),
            pltpu.SemaphoreType.DMA((2, H)),
            pltpu.SemaphoreType.DMA((N_DEV - 1,)),
            pltpu.SemaphoreType.DMA((N_DEV - 1,)),
        ],
        compiler_params=pltpu.CompilerParams(
            collective_id=0, vmem_limit_bytes=64 * 1024 * 1024),
    )(Q, kt, vt, bt, lens2)


# device time: 27276 ns/iter; 3.3009x vs baseline; 1.1159x over previous
import jax
import jax.numpy as jnp
from jax import lax
from jax.experimental import pallas as pl
from jax.experimental.pallas import tpu as pltpu

N_DEV = 8
B = 16
NB = 128
BS = 16
H = 16
D = 64
PAGES = 128
TOK = PAGES * BS
NEG = -1e30


def kernel(Q, K, V, bt, lens):
    lens2 = lens.reshape(1, B)
    kt = K.transpose(1, 2, 3, 0)
    vt = V.transpose(1, 2, 3, 0)

    def body(q_ref, k_ref, v_ref, bt_ref, lens_ref, out_ref,
             kbuf, vbuf, gbuf, kv_sems, send_sems, recv_sems):
        my_pos = lax.axis_index("i")
        my_lo = my_pos * PAGES

        kdmas = [[None] * BS for _ in range(H)]
        vdmas = [[None] * BS for _ in range(H)]
        for h in range(H):
            for t in range(BS):
                lanes = pl.ds(t * PAGES, PAGES)
                kd = pltpu.make_async_copy(
                    k_ref.at[t, h], kbuf.at[h, :, lanes], kv_sems.at[0, h])
                vd = pltpu.make_async_copy(
                    v_ref.at[t, h], vbuf.at[h, :, lanes], kv_sems.at[1, h])
                kd.start()
                vd.start()
                kdmas[h][t] = kd
                vdmas[h][t] = vd

        barrier_sem = pltpu.get_barrier_semaphore()
        for k in range(1, N_DEV):
            pl.semaphore_signal(
                barrier_sem, inc=1,
                device_id=(lax.rem(my_pos + k, N_DEV),),
                device_id_type=pl.DeviceIdType.MESH,
            )

        btv = bt_ref[...]
        ln = jnp.transpose(lens_ref[...], (1, 0))
        slot = lax.broadcasted_iota(jnp.int32, (B, NB), 1)
        btv = jnp.where(slot < ln, btv, -1)
        page_ids = my_lo + lax.broadcasted_iota(jnp.int32, (B, NB, PAGES), 2)
        w = jnp.sum((btv[:, :, None] == page_ids).astype(jnp.float32), axis=1)
        w_tok = jnp.broadcast_to(w[:, None, :], (B, BS, PAGES)).reshape(B, TOK)
        has_key = w_tok > 0.0

        scale = D ** -0.5

        ms, ls = [], []
        for h in range(H):
            q_h = q_ref[:, 0, h, :]
            for t in range(BS):
                kdmas[h][t].wait()
            k_h = kbuf[h]
            s = lax.dot_general(
                q_h, k_h, (((1,), (0,)), ((), ())),
                preferred_element_type=jnp.float32,
            ) * scale
            s = jnp.where(has_key, s, NEG)
            m_h = jnp.max(s, axis=1, keepdims=True)
            p_h = jnp.exp(s - m_h) * w_tok
            l_h = jnp.sum(p_h, axis=1, keepdims=True)
            for t in range(BS):
                vdmas[h][t].wait()
            v_h = vbuf[h]
            acc_h = lax.dot_general(
                p_h, v_h, (((1,), (1,)), ((), ())),
                preferred_element_type=jnp.float32,
            )
            gbuf[my_pos, h] = acc_h.astype(jnp.bfloat16)
            ms.append(m_h)
            ls.append(l_h)
        stats = jnp.concatenate(
            ms + ls + [jnp.zeros((B, D - 2 * H), jnp.float32)], axis=1)
        gbuf[my_pos, H] = stats.astype(jnp.bfloat16)

        pl.semaphore_wait(barrier_sem, N_DEV - 1)

        rdmas = []
        for k in range(1, N_DEV):
            rdma = pltpu.make_async_remote_copy(
                src_ref=gbuf.at[my_pos],
                dst_ref=gbuf.at[my_pos],
                send_sem=send_sems.at[k - 1],
                recv_sem=recv_sems.at[k - 1],
                device_id=(lax.rem(my_pos + k, N_DEV),),
                device_id_type=pl.DeviceIdType.MESH,
            )
            rdma.start()
            rdmas.append(rdma)
        for rdma in rdmas:
            rdma.wait()

        g = gbuf[...].astype(jnp.float32)
        mT = g[:, H, :, 0:H]
        lT = g[:, H, :, H:2 * H]
        mx = jnp.max(mT, axis=0)
        sc = jnp.exp(mT - mx[None])
        den = jnp.sum(sc * lT, axis=0)
        scT = jnp.transpose(sc, (0, 2, 1))
        num = jnp.sum(scT[..., None] * g[:, :H], axis=0)
        res = num / jnp.transpose(den, (1, 0))[..., None]
        out_ref[:, 0, :, :] = jnp.transpose(res, (1, 0, 2))

    return pl.pallas_call(
        body,
        out_shape=jax.ShapeDtypeStruct((B, 1, H, D), jnp.float32),
        in_specs=[
            pl.BlockSpec(memory_space=pltpu.VMEM),
            pl.BlockSpec(memory_space=pltpu.HBM),
            pl.BlockSpec(memory_space=pltpu.HBM),
            pl.BlockSpec(memory_space=pltpu.VMEM),
            pl.BlockSpec(memory_space=pltpu.VMEM),
        ],
        out_specs=pl.BlockSpec(memory_space=pltpu.VMEM),
        scratch_shapes=[
            pltpu.VMEM((H, D, TOK), jnp.float32),
            pltpu.VMEM((H, D, TOK), jnp.float32),
            pltpu.VMEM((N_DEV, H + 1, B, D), jnp.bfloat16),
            pltpu.SemaphoreType.DMA((2, H)),
            pltpu.SemaphoreType.DMA((N_DEV - 1,)),
            pltpu.SemaphoreType.DMA((N_DEV - 1,)),
        ],
        compiler_params=pltpu.CompilerParams(
            collective_id=0, vmem_limit_bytes=64 * 1024 * 1024),
    )(Q, kt, vt, bt, lens2)


# device time: 26294 ns/iter; 3.4242x vs baseline; 1.0373x over previous
import jax
import jax.numpy as jnp
from jax import lax
from jax.experimental import pallas as pl
from jax.experimental.pallas import tpu as pltpu

N_DEV = 8
B = 16
NB = 128
BS = 16
H = 16
D = 64
PAGES = 128
TOK = PAGES * BS
HH = H // 2
NEG = -1e30


def kernel(Q, K, V, bt, lens):
    lens2 = lens.reshape(1, B)
    kt = K.transpose(1, 2, 3, 0)
    vt = V.transpose(1, 2, 3, 0)

    def body(q_ref, k_ref, v_ref, bt_ref, lens_ref, out_ref,
             kbuf, vbuf, gbuf, kv_sems, send_sems, recv_sems):
        my_pos = lax.axis_index("i")
        my_lo = my_pos * PAGES

        kdmas = [[None] * BS for _ in range(H)]
        vdmas = [[None] * BS for _ in range(H)]
        for h in range(H):
            for t in range(BS):
                lanes = pl.ds(t * PAGES, PAGES)
                kd = pltpu.make_async_copy(
                    k_ref.at[t, h], kbuf.at[h, :, lanes], kv_sems.at[0, h])
                vd = pltpu.make_async_copy(
                    v_ref.at[t, h], vbuf.at[h, :, lanes], kv_sems.at[1, h])
                kd.start()
                vd.start()
                kdmas[h][t] = kd
                vdmas[h][t] = vd

        barrier_sem = pltpu.get_barrier_semaphore()
        for k in range(1, N_DEV):
            pl.semaphore_signal(
                barrier_sem, inc=1,
                device_id=(lax.rem(my_pos + k, N_DEV),),
                device_id_type=pl.DeviceIdType.MESH,
            )

        btv = bt_ref[...]
        ln = jnp.transpose(lens_ref[...], (1, 0))
        slot = lax.broadcasted_iota(jnp.int32, (B, NB), 1)
        btv = jnp.where(slot < ln, btv, -1)
        page_ids = my_lo + lax.broadcasted_iota(jnp.int32, (B, NB, PAGES), 2)
        w = jnp.sum((btv[:, :, None] == page_ids).astype(jnp.float32), axis=1)
        w_tok = jnp.broadcast_to(w[:, None, :], (B, BS, PAGES)).reshape(B, TOK)
        has_key = w_tok > 0.0

        scale = D ** -0.5

        rdmas = []
        for r in range(2):
            ms, ls = [], []
            for hh in range(HH):
                h = r * HH + hh
                q_h = q_ref[:, 0, h, :]
                for t in range(BS):
                    kdmas[h][t].wait()
                k_h = kbuf[h]
                s = lax.dot_general(
                    q_h, k_h, (((1,), (0,)), ((), ())),
                    preferred_element_type=jnp.float32,
                ) * scale
                s = jnp.where(has_key, s, NEG)
                m_h = jnp.max(s, axis=1, keepdims=True)
                p_h = jnp.exp(s - m_h) * w_tok
                l_h = jnp.sum(p_h, axis=1, keepdims=True)
                for t in range(BS):
                    vdmas[h][t].wait()
                v_h = vbuf[h]
                acc_h = lax.dot_general(
                    p_h, v_h, (((1,), (1,)), ((), ())),
                    preferred_element_type=jnp.float32,
                )
                gbuf[my_pos, r, hh] = acc_h.astype(jnp.bfloat16)
                ms.append(m_h)
                ls.append(l_h)
            stats = jnp.concatenate(
                ms + ls + [jnp.zeros((B, D - 2 * HH), jnp.float32)], axis=1)
            gbuf[my_pos, r, HH] = stats.astype(jnp.bfloat16)

            if r == 0:
                pl.semaphore_wait(barrier_sem, N_DEV - 1)
            for k in range(1, N_DEV):
                rdma = pltpu.make_async_remote_copy(
                    src_ref=gbuf.at[my_pos, r],
                    dst_ref=gbuf.at[my_pos, r],
                    send_sem=send_sems.at[r, k - 1],
                    recv_sem=recv_sems.at[r, k - 1],
                    device_id=(lax.rem(my_pos + k, N_DEV),),
                    device_id_type=pl.DeviceIdType.MESH,
                )
                rdma.start()
                rdmas.append(rdma)
        for rdma in rdmas:
            rdma.wait()

        g = gbuf[...].astype(jnp.float32)
        mT = jnp.concatenate(
            [g[:, 0, HH, :, 0:HH], g[:, 1, HH, :, 0:HH]], axis=-1)
        lT = jnp.concatenate(
            [g[:, 0, HH, :, HH:2 * HH], g[:, 1, HH, :, HH:2 * HH]], axis=-1)
        gacc = jnp.concatenate(
            [g[:, 0, :HH], g[:, 1, :HH]], axis=1)
        mx = jnp.max(mT, axis=0)
        sc = jnp.exp(mT - mx[None])
        den = jnp.sum(sc * lT, axis=0)
        scT = jnp.transpose(sc, (0, 2, 1))
        num = jnp.sum(scT[..., None] * gacc, axis=0)
        res = num / jnp.transpose(den, (1, 0))[..., None]
        out_ref[:, 0, :, :] = jnp.transpose(res, (1, 0, 2))

    return pl.pallas_call(
        body,
        out_shape=jax.ShapeDtypeStruct((B, 1, H, D), jnp.float32),
        in_specs=[
            pl.BlockSpec(memory_space=pltpu.VMEM),
            pl.BlockSpec(memory_space=pltpu.HBM),
            pl.BlockSpec(memory_space=pltpu.HBM),
            pl.BlockSpec(memory_space=pltpu.VMEM),
            pl.BlockSpec(memory_space=pltpu.VMEM),
        ],
        out_specs=pl.BlockSpec(memory_space=pltpu.VMEM),
        scratch_shapes=[
            pltpu.VMEM((H, D, TOK), jnp.float32),
            pltpu.VMEM((H, D, TOK), jnp.float32),
            pltpu.VMEM((N_DEV, 2, HH + 1, B, D), jnp.bfloat16),
            pltpu.SemaphoreType.DMA((2, H)),
            pltpu.SemaphoreType.DMA((2, N_DEV - 1)),
            pltpu.SemaphoreType.DMA((2, N_DEV - 1)),
        ],
        compiler_params=pltpu.CompilerParams(
            collective_id=0, vmem_limit_bytes=64 * 1024 * 1024),
    )(Q, kt, vt, bt, lens2)


# device time: 26180 ns/iter; 3.4391x vs baseline; 1.0044x over previous
import jax
import jax.numpy as jnp
from jax import lax
from jax.experimental import pallas as pl
from jax.experimental.pallas import tpu as pltpu

N_DEV = 8
B = 16
NB = 128
BS = 16
H = 16
D = 64
PAGES = 128
TOK = PAGES * BS
NR = 4
HH = H // NR
NEG = -1e30


def kernel(Q, K, V, bt, lens):
    lens2 = lens.reshape(1, B)
    kt = K.transpose(1, 2, 3, 0)
    vt = V.transpose(1, 2, 3, 0)

    def body(q_ref, k_ref, v_ref, bt_ref, lens_ref, out_ref,
             kbuf, vbuf, gbuf, kv_sems, send_sems, recv_sems):
        my_pos = lax.axis_index("i")
        my_lo = my_pos * PAGES

        kdmas = [[None] * BS for _ in range(H)]
        vdmas = [[None] * BS for _ in range(H)]
        for h in range(H):
            for t in range(BS):
                lanes = pl.ds(t * PAGES, PAGES)
                kd = pltpu.make_async_copy(
                    k_ref.at[t, h], kbuf.at[h, :, lanes], kv_sems.at[0, h])
                vd = pltpu.make_async_copy(
                    v_ref.at[t, h], vbuf.at[h, :, lanes], kv_sems.at[1, h])
                kd.start()
                vd.start()
                kdmas[h][t] = kd
                vdmas[h][t] = vd

        barrier_sem = pltpu.get_barrier_semaphore()
        for k in range(1, N_DEV):
            pl.semaphore_signal(
                barrier_sem, inc=1,
                device_id=(lax.rem(my_pos + k, N_DEV),),
                device_id_type=pl.DeviceIdType.MESH,
            )

        btv = bt_ref[...]
        ln = jnp.transpose(lens_ref[...], (1, 0))
        slot = lax.broadcasted_iota(jnp.int32, (B, NB), 1)
        btv = jnp.where(slot < ln, btv, -1)
        page_ids = my_lo + lax.broadcasted_iota(jnp.int32, (B, NB, PAGES), 2)
        w = jnp.sum((btv[:, :, None] == page_ids).astype(jnp.float32), axis=1)
        w_tok = jnp.broadcast_to(w[:, None, :], (B, BS, PAGES)).reshape(B, TOK)
        has_key = w_tok > 0.0

        scale = D ** -0.5

        rdmas = []
        for r in range(NR):
            ms, ls = [], []
            for hh in range(HH):
                h = r * HH + hh
                q_h = q_ref[:, 0, h, :]
                for t in range(BS):
                    kdmas[h][t].wait()
                k_h = kbuf[h]
                s = lax.dot_general(
                    q_h, k_h, (((1,), (0,)), ((), ())),
                    preferred_element_type=jnp.float32,
                ) * scale
                s = jnp.where(has_key, s, NEG)
                m_h = jnp.max(s, axis=1, keepdims=True)
                p_h = jnp.exp(s - m_h) * w_tok
                l_h = jnp.sum(p_h, axis=1, keepdims=True)
                for t in range(BS):
                    vdmas[h][t].wait()
                v_h = vbuf[h]
                acc_h = lax.dot_general(
                    p_h, v_h, (((1,), (1,)), ((), ())),
                    preferred_element_type=jnp.float32,
                )
                gbuf[my_pos, r, hh] = acc_h.astype(jnp.bfloat16)
                ms.append(m_h)
                ls.append(l_h)
            stats = jnp.concatenate(
                ms + ls + [jnp.zeros((B, D - 2 * HH), jnp.float32)], axis=1)
            gbuf[my_pos, r, HH] = stats.astype(jnp.bfloat16)

            if r == 0:
                pl.semaphore_wait(barrier_sem, N_DEV - 1)
            for k in range(1, N_DEV):
                rdma = pltpu.make_async_remote_copy(
                    src_ref=gbuf.at[my_pos, r],
                    dst_ref=gbuf.at[my_pos, r],
                    send_sem=send_sems.at[r, k - 1],
                    recv_sem=recv_sems.at[r, k - 1],
                    device_id=(lax.rem(my_pos + k, N_DEV),),
                    device_id_type=pl.DeviceIdType.MESH,
                )
                rdma.start()
                rdmas.append(rdma)
        for rdma in rdmas:
            rdma.wait()

        g = gbuf[...].astype(jnp.float32)
        mT = jnp.concatenate(
            [g[:, r, HH, :, 0:HH] for r in range(NR)], axis=-1)
        lT = jnp.concatenate(
            [g[:, r, HH, :, HH:2 * HH] for r in range(NR)], axis=-1)
        gacc = jnp.concatenate(
            [g[:, r, :HH] for r in range(NR)], axis=1)
        mx = jnp.max(mT, axis=0)
        sc = jnp.exp(mT - mx[None])
        den = jnp.sum(sc * lT, axis=0)
        scT = jnp.transpose(sc, (0, 2, 1))
        num = jnp.sum(scT[..., None] * gacc, axis=0)
        res = num / jnp.transpose(den, (1, 0))[..., None]
        out_ref[:, 0, :, :] = jnp.transpose(res, (1, 0, 2))

    return pl.pallas_call(
        body,
        out_shape=jax.ShapeDtypeStruct((B, 1, H, D), jnp.float32),
        in_specs=[
            pl.BlockSpec(memory_space=pltpu.VMEM),
            pl.BlockSpec(memory_space=pltpu.HBM),
            pl.BlockSpec(memory_space=pltpu.HBM),
            pl.BlockSpec(memory_space=pltpu.VMEM),
            pl.BlockSpec(memory_space=pltpu.VMEM),
        ],
        out_specs=pl.BlockSpec(memory_space=pltpu.VMEM),
        scratch_shapes=[
            pltpu.VMEM((H, D, TOK), jnp.float32),
            pltpu.VMEM((H, D, TOK), jnp.float32),
            pltpu.VMEM((N_DEV, NR, HH + 1, B, D), jnp.bfloat16),
            pltpu.SemaphoreType.DMA((2, H)),
            pltpu.SemaphoreType.DMA((NR, N_DEV - 1)),
            pltpu.SemaphoreType.DMA((NR, N_DEV - 1)),
        ],
        compiler_params=pltpu.CompilerParams(
            collective_id=0, vmem_limit_bytes=64 * 1024 * 1024),
    )(Q, kt, vt, bt, lens2)
